# baseline (speedup 1.0000x reference)
import functools

import jax
import jax.numpy as jnp
from jax import lax
from jax.experimental import pallas as pl
from jax.experimental.pallas import tpu as pltpu
from jax.experimental.pallas import tpu_sc as plsc

NUM_CORES = 2
NUM_SUBCORES = 16
NW = NUM_CORES * NUM_SUBCORES
LANES = 16

B = 16384
D = 64
BPW = B // NW
NGROUP = BPW // LANES
BURST = 4
NRING = 2 * BURST
PD = 128

_CP = pltpu.CompilerParams(needs_layout_passes=False, use_tc_tiling_on_sc=True)
_MESH = dict(core_axis_name="c", subcore_axis_name="s")


def _cond_fetch_chain(vids, prev, fcnt, tbl_hbm, slab_v, sem):
    blks, slots, changed = [], [], []
    prev_blk, prev_slot = prev
    for kk in vids:
        blk = kk >> 7
        ch = blk != prev_blk
        slot = jnp.where(ch, fcnt & (NRING - 1), prev_slot)
        fcnt = fcnt + ch.astype(jnp.int32)
        blks.append(blk)
        slots.append(slot)
        changed.append(ch)
        prev_blk, prev_slot = blk, slot
    for e, (blk, ch) in enumerate(zip(blks, changed)):

        @pl.when(ch)
        def _(blk=blk, e=e):
            off = pl.multiple_of(blk * 128, 128)
            pltpu.async_copy(
                tbl_hbm.at[:, pl.ds(off, 128)], slab_v.at[slots[e]], sem)

    return blks, slots, changed, (prev_blk, prev_slot), fcnt


def _cond_fetch_drain(blks, slots, changed, tbl_hbm, slab_v, sem):
    for e, (blk, ch) in enumerate(zip(blks, changed)):

        @pl.when(ch)
        def _(blk=blk, e=e):
            off = pl.multiple_of(blk * 128, 128)
            pltpu.make_async_copy(
                tbl_hbm.at[:, pl.ds(off, 128)], slab_v.at[slots[e]],
                sem).wait()


def _phase1(users2, ut_t):
    mesh = plsc.VectorSubcoreMesh(**_MESH)

    @functools.partial(
        pl.kernel,
        mesh=mesh,
        out_type=jax.ShapeDtypeStruct((B, PD), jnp.float32),
        compiler_params=_CP,
        scratch_types=[
            pltpu.VMEM((BPW,), jnp.int32),
            pltpu.VMEM((NRING, D, 128), jnp.float32),
            pltpu.VMEM((128, PD), jnp.float32),
            pltpu.SemaphoreType.DMA,
        ],
    )
    def k1(u_hbm, ut_hbm, uemb_hbm, uidx_v, uslab_v, rows_v, sem):
        wid = lax.axis_index("s") * NUM_CORES + lax.axis_index("c")
        pltpu.sync_copy(u_hbm.at[wid], uidx_v)
        lanes = lax.iota(jnp.int32, LANES)

        def round_body(g, carry):
            prev, fcnt = carry[:2], carry[2]
            vu = uidx_v[pl.ds(g * LANES, LANES)]
            for sub in range(LANES // BURST):
                vids = [vu[sub * BURST + e] for e in range(BURST)]
                blks, slots, changed, prev, fcnt = _cond_fetch_chain(
                    vids, prev, fcnt, ut_hbm, uslab_v, sem)
                _cond_fetch_drain(blks, slots, changed, ut_hbm, uslab_v, sem)
                for e in range(BURST):
                    kk = sub * BURST + e
                    ucol = jnp.full((LANES,), vids[e] & 127, jnp.int32)
                    uslb = jnp.full((LANES,), slots[e], jnp.int32)
                    rloc = jnp.full(
                        (LANES,), (g & 7) * LANES + kk, jnp.int32)
                    for c in range(D // LANES):
                        dvec = lanes + c * LANES
                        gu = plsc.load_gather(uslab_v, [uslb, dvec, ucol])
                        plsc.store_scatter(rows_v, [rloc, dvec], gu)

            @pl.when((g & 7) == 7)
            def _():
                off = pl.multiple_of(wid * BPW + (g >> 3) * 128, 128)
                pltpu.sync_copy(rows_v, uemb_hbm.at[pl.ds(off, 128), :])

            return prev[0], prev[1], fcnt

        lax.fori_loop(0, NGROUP, round_body,
                      (jnp.int32(-1), jnp.int32(0), jnp.int32(0)))

    return k1(users2, ut_t)


def _phase2(items2, ku3, perm2, it_t, uemb):
    mesh = plsc.VectorSubcoreMesh(**_MESH)

    @functools.partial(
        pl.kernel,
        mesh=mesh,
        out_type=jax.ShapeDtypeStruct((B,), jnp.float32),
        compiler_params=_CP,
        scratch_types=[
            pltpu.VMEM((BPW,), jnp.int32),
            pltpu.VMEM((4, 128), jnp.int32),
            pltpu.VMEM((BPW,), jnp.int32),
            pltpu.VMEM((NRING, D, 128), jnp.float32),
            pltpu.VMEM((128, PD), jnp.float32),
            pltpu.VMEM((BPW,), jnp.float32),
            pltpu.SemaphoreType.DMA,
            pltpu.SemaphoreType.DMA,
        ],
    )
    def k2(i_hbm, ku_hbm, p_hbm, it_hbm, uemb_hbm, out_hbm,
           iidx_v, kuidx_v, pidx_v, islab_v, urows_v, sc_v, sem, osem):
        wid = lax.axis_index("s") * NUM_CORES + lax.axis_index("c")
        pltpu.sync_copy(i_hbm.at[wid], iidx_v)
        pltpu.sync_copy(ku_hbm.at[wid], kuidx_v)
        pltpu.sync_copy(p_hbm.at[wid], pidx_v)
        lanes = lax.iota(jnp.int32, LANES)

        carry = (jnp.int32(-1), jnp.int32(0), jnp.int32(0))
        for q in range(4):
            pltpu.async_copy(
                uemb_hbm.at[kuidx_v.at[q]], urows_v, sem).wait()

            def round_body(g, carry, q=q):
                prev, fcnt = carry[:2], carry[2]
                vi = iidx_v[pl.ds(g * LANES, LANES)]
                dots = jnp.zeros((LANES,), jnp.float32)
                for sub in range(LANES // BURST):
                    vids = [vi[sub * BURST + e] for e in range(BURST)]
                    blks, slots, changed, prev, fcnt = _cond_fetch_chain(
                        vids, prev, fcnt, it_hbm, islab_v, sem)
                    _cond_fetch_drain(
                        blks, slots, changed, it_hbm, islab_v, sem)
                    for e in range(BURST):
                        kk = sub * BURST + e
                        icol = jnp.full((LANES,), vids[e] & 127, jnp.int32)
                        islb = jnp.full((LANES,), slots[e], jnp.int32)
                        rloc = jnp.full(
                            (LANES,), (g & 7) * LANES + kk, jnp.int32)
                        acc = jnp.zeros((LANES,), jnp.float32)
                        for c in range(D // LANES):
                            dvec = lanes + c * LANES
                            gi = plsc.load_gather(
                                islab_v, [islb, dvec, icol])
                            gu = plsc.load_gather(urows_v, [rloc, dvec])
                            acc = acc + gu * gi
                        dots = jnp.where(lanes == kk, jnp.sum(acc), dots)
                sc_v[pl.ds(g * LANES, LANES)] = 1.0 / (1.0 + jnp.exp(-dots))
                return prev[0], prev[1], fcnt

            carry = lax.fori_loop(q * 8, (q + 1) * 8, round_body, carry)

        pltpu.async_copy(sc_v, out_hbm.at[pidx_v], osem).wait()

    return k2(items2, ku3, perm2, it_t, uemb)


def kernel(users, items, user_table, item_table):
    iota = lax.iota(jnp.int32, B)
    users_s, pu = lax.sort((users, iota), dimension=0, num_keys=1)
    items_s, pi = lax.sort((items, iota), dimension=0, num_keys=1)
    inv_pu = jnp.argsort(pu)
    ku = inv_pu[pi]
    uemb = _phase1(users_s.reshape(NW, BPW), user_table.T)
    return _phase2(
        items_s.reshape(NW, BPW),
        ku.reshape(NW, 4, 128),
        pi.reshape(NW, BPW),
        item_table.T,
        uemb)


# speedup vs baseline: 1.2665x; 1.2665x over previous
import functools

import jax
import jax.numpy as jnp
from jax import lax
from jax.experimental import pallas as pl
from jax.experimental.pallas import tpu as pltpu
from jax.experimental.pallas import tpu_sc as plsc

NUM_CORES = 2
NUM_SUBCORES = 16
NW = NUM_CORES * NUM_SUBCORES
LANES = 16

B = 16384
D = 64
BPW = B // NW
NGROUP = BPW // LANES
BURST = 4
NSUB = LANES // BURST
NRING = 3 * BURST
PD = 128

_CP = pltpu.CompilerParams(needs_layout_passes=False, use_tc_tiling_on_sc=True)
_MESH = dict(core_axis_name="c", subcore_axis_name="s")


def _cond_fetch_chain(vids, prev, fcnt, tbl_hbm, slab_v, sem):
    blks, slots, changed = [], [], []
    prev_blk, prev_slot = prev
    for kk in vids:
        blk = kk >> 7
        ch = blk != prev_blk
        slot = jnp.where(ch, fcnt, prev_slot)
        fcnt = jnp.where(ch, jnp.where(fcnt == NRING - 1, 0, fcnt + 1), fcnt)
        blks.append(blk)
        slots.append(slot)
        changed.append(ch)
        prev_blk, prev_slot = blk, slot
    for e, (blk, ch) in enumerate(zip(blks, changed)):

        @pl.when(ch)
        def _(blk=blk, e=e):
            off = pl.multiple_of(blk * 128, 128)
            pltpu.async_copy(
                tbl_hbm.at[:, pl.ds(off, 128)], slab_v.at[slots[e]], sem)

    return blks, slots, changed, (prev_blk, prev_slot), fcnt


def _cond_fetch_drain(blks, slots, changed, tbl_hbm, slab_v, sem):
    for e, (blk, ch) in enumerate(zip(blks, changed)):

        @pl.when(ch)
        def _(blk=blk, e=e):
            off = pl.multiple_of(blk * 128, 128)
            pltpu.make_async_copy(
                tbl_hbm.at[:, pl.ds(off, 128)], slab_v.at[slots[e]],
                sem).wait()


def _pipelined_subs(vvec, prev, fcnt, tbl_hbm, slab_v, sems, extract):
    pend = None
    for sub in range(NSUB):
        vids = [vvec[sub * BURST + e] for e in range(BURST)]
        blks, slots, changed, prev, fcnt = _cond_fetch_chain(
            vids, prev, fcnt, tbl_hbm, slab_v, sems[sub % 2])
        if pend is not None:
            psub, pvids, pblks, pslots, pchanged = pend
            _cond_fetch_drain(pblks, pslots, pchanged, tbl_hbm, slab_v,
                              sems[psub % 2])
            extract(psub, pvids, pslots)
        pend = (sub, vids, blks, slots, changed)
    psub, pvids, pblks, pslots, pchanged = pend
    _cond_fetch_drain(pblks, pslots, pchanged, tbl_hbm, slab_v,
                      sems[psub % 2])
    extract(psub, pvids, pslots)
    return prev, fcnt


def _phase1(users2, ut_t):
    mesh = plsc.VectorSubcoreMesh(**_MESH)

    @functools.partial(
        pl.kernel,
        mesh=mesh,
        out_type=jax.ShapeDtypeStruct((B, PD), jnp.float32),
        compiler_params=_CP,
        scratch_types=[
            pltpu.VMEM((BPW,), jnp.int32),
            pltpu.VMEM((NRING, D, 128), jnp.float32),
            pltpu.VMEM((128, PD), jnp.float32),
            pltpu.SemaphoreType.DMA,
            pltpu.SemaphoreType.DMA,
        ],
    )
    def k1(u_hbm, ut_hbm, uemb_hbm, uidx_v, uslab_v, rows_v, sem, semb):
        wid = lax.axis_index("s") * NUM_CORES + lax.axis_index("c")
        pltpu.sync_copy(u_hbm.at[wid], uidx_v)
        lanes = lax.iota(jnp.int32, LANES)

        def round_body(g, carry):
            prev, fcnt = carry[:2], carry[2]
            vu = uidx_v[pl.ds(g * LANES, LANES)]

            def extract(sub, vids, slots):
                for e in range(BURST):
                    kk = sub * BURST + e
                    ucol = jnp.full((LANES,), vids[e] & 127, jnp.int32)
                    uslb = jnp.full((LANES,), slots[e], jnp.int32)
                    rloc = jnp.full(
                        (LANES,), (g & 7) * LANES + kk, jnp.int32)
                    for c in range(D // LANES):
                        dvec = lanes + c * LANES
                        gu = plsc.load_gather(uslab_v, [uslb, dvec, ucol])
                        plsc.store_scatter(rows_v, [rloc, dvec], gu)

            prev, fcnt = _pipelined_subs(
                vu, prev, fcnt, ut_hbm, uslab_v, (sem, semb), extract)

            @pl.when((g & 7) == 7)
            def _():
                off = pl.multiple_of(wid * BPW + (g >> 3) * 128, 128)
                pltpu.sync_copy(rows_v, uemb_hbm.at[pl.ds(off, 128), :])

            return prev[0], prev[1], fcnt

        lax.fori_loop(0, NGROUP, round_body,
                      (jnp.int32(-1), jnp.int32(0), jnp.int32(0)))

    return k1(users2, ut_t)


def _phase2(items2, ku3, perm2, it_t, uemb):
    mesh = plsc.VectorSubcoreMesh(**_MESH)

    @functools.partial(
        pl.kernel,
        mesh=mesh,
        out_type=jax.ShapeDtypeStruct((B,), jnp.float32),
        compiler_params=_CP,
        scratch_types=[
            pltpu.VMEM((BPW,), jnp.int32),
            pltpu.VMEM((4, 128), jnp.int32),
            pltpu.VMEM((BPW,), jnp.int32),
            pltpu.VMEM((NRING, D, 128), jnp.float32),
            pltpu.VMEM((128, PD), jnp.float32),
            pltpu.VMEM((BPW,), jnp.float32),
            pltpu.SemaphoreType.DMA,
            pltpu.SemaphoreType.DMA,
            pltpu.SemaphoreType.DMA,
        ],
    )
    def k2(i_hbm, ku_hbm, p_hbm, it_hbm, uemb_hbm, out_hbm,
           iidx_v, kuidx_v, pidx_v, islab_v, urows_v, sc_v, sem, semb, osem):
        wid = lax.axis_index("s") * NUM_CORES + lax.axis_index("c")
        pltpu.sync_copy(i_hbm.at[wid], iidx_v)
        pltpu.sync_copy(ku_hbm.at[wid], kuidx_v)
        pltpu.sync_copy(p_hbm.at[wid], pidx_v)
        lanes = lax.iota(jnp.int32, LANES)

        carry = (jnp.int32(-1), jnp.int32(0), jnp.int32(0))
        for q in range(4):
            pltpu.async_copy(
                uemb_hbm.at[kuidx_v.at[q]], urows_v, sem).wait()

            def round_body(g, carry, q=q):
                prev, fcnt = carry[:2], carry[2]
                vi = iidx_v[pl.ds(g * LANES, LANES)]
                dots_h = [jnp.zeros((LANES,), jnp.float32)]

                def extract(sub, vids, slots):
                    for e in range(BURST):
                        kk = sub * BURST + e
                        icol = jnp.full((LANES,), vids[e] & 127, jnp.int32)
                        islb = jnp.full((LANES,), slots[e], jnp.int32)
                        rloc = jnp.full(
                            (LANES,), (g & 7) * LANES + kk, jnp.int32)
                        acc = jnp.zeros((LANES,), jnp.float32)
                        for c in range(D // LANES):
                            dvec = lanes + c * LANES
                            gi = plsc.load_gather(
                                islab_v, [islb, dvec, icol])
                            gu = plsc.load_gather(urows_v, [rloc, dvec])
                            acc = acc + gu * gi
                        dots_h[0] = jnp.where(
                            lanes == kk, jnp.sum(acc), dots_h[0])

                prev, fcnt = _pipelined_subs(
                    vi, prev, fcnt, it_hbm, islab_v, (sem, semb), extract)
                sc_v[pl.ds(g * LANES, LANES)] = (
                    1.0 / (1.0 + jnp.exp(-dots_h[0])))
                return prev[0], prev[1], fcnt

            carry = lax.fori_loop(q * 8, (q + 1) * 8, round_body, carry)

        pltpu.async_copy(sc_v, out_hbm.at[pidx_v], osem).wait()

    return k2(items2, ku3, perm2, it_t, uemb)


def kernel(users, items, user_table, item_table):
    iota = lax.iota(jnp.int32, B)
    users_s, pu = lax.sort((users, iota), dimension=0, num_keys=1)
    items_s, pi = lax.sort((items, iota), dimension=0, num_keys=1)
    inv_pu = jnp.argsort(pu)
    ku = inv_pu[pi]
    uemb = _phase1(users_s.reshape(NW, BPW), user_table.T)
    return _phase2(
        items_s.reshape(NW, BPW),
        ku.reshape(NW, 4, 128),
        pi.reshape(NW, BPW),
        item_table.T,
        uemb)
